# baseline (speedup 1.0000x reference)
import math

import jax
import jax.numpy as jnp
from jax.experimental import pallas as pl
from jax.experimental.pallas import tpu as pltpu

DIM = 1024
HALF = DIM // 2
MAX_LEN = 5000
MID_POS = MAX_LEN // 2
BLK = 1024
SUB = 512


def _pe_kernel(shift_ref, emb_ref, dpe_ref, ape_ref, out_ref):
    i = pl.program_id(0)
    b = pl.program_id(1)
    blk_start = i * BLK

    s0 = shift_ref[b, 0]
    s1 = shift_ref[b, 1]
    s2 = shift_ref[b, 2]
    s3 = shift_ref[b, 3]
    m0 = (s0 + s1) // 2 + 1
    m1 = (s1 + s2) // 2 + 1
    m2 = (s2 + s3) // 2 + 1

    def seg_of(p):
        return (
            (p >= m0).astype(jnp.int32)
            + (p >= m1).astype(jnp.int32)
            + (p >= m2).astype(jnp.int32)
        )

    for t in range(BLK // SUB):
        rows = slice(t * SUB, (t + 1) * SUB)
        sub_start = blk_start + t * SUB
        base = MID_POS + sub_start
        j_lo = seg_of(sub_start)
        j_hi = seg_of(sub_start + SUB - 1)

        emb1 = emb_ref[0, rows, :HALF]
        second = emb_ref[0, rows, HALF:] + ape_ref[0, rows, :]

        def load_big(s):
            a = base - s
            a0 = jax.lax.div(a, 8) * 8
            return dpe_ref[pl.ds(a0, SUB + 8), :], a - a0

        @pl.when(j_lo == j_hi)
        def _single_segment(emb1=emb1, second=second, rows=rows,
                            j_lo=j_lo, load_big=load_big):
            s = shift_ref[b, j_lo]
            big, r = load_big(s)
            for rr in range(8):

                @pl.when(r == rr)
                def _(rr=rr):
                    first = emb1 + big[rr : rr + SUB]
                    out_ref[0, rows, :] = jnp.concatenate([first, second], -1)

        @pl.when(j_lo != j_hi)
        def _multi_segment(emb1=emb1, second=second, rows=rows,
                           sub_start=sub_start, load_big=load_big):
            pos = sub_start + jax.lax.broadcasted_iota(jnp.int32, (SUB, 1), 0)
            seg = seg_of(pos)
            n = SUB + 8

            def load_chunk(s):
                big, r = load_big(s)
                return pltpu.roll(big, jax.lax.rem(n - r, n), 0)[:SUB]

            sel = load_chunk(s0)
            for j, s in ((1, s1), (2, s2), (3, s3)):
                sel = jnp.where(seg == j, load_chunk(s), sel)
            out_ref[0, rows, :] = jnp.concatenate([emb1 + sel, second], -1)


def kernel(emb, shift, dpe, ape):
    b, length, d = emb.shape
    nblk = length // BLK
    grid = (nblk, b)
    return pl.pallas_call(
        _pe_kernel,
        grid=grid,
        in_specs=[
            pl.BlockSpec(memory_space=pltpu.SMEM),
            pl.BlockSpec((1, BLK, DIM), lambda i, b_: (b_, i, 0)),
            pl.BlockSpec((MAX_LEN, HALF), lambda i, b_: (0, 0)),
            pl.BlockSpec((1, BLK, HALF), lambda i, b_: (0, i, 0)),
        ],
        out_specs=pl.BlockSpec((1, BLK, DIM), lambda i, b_: (b_, i, 0)),
        out_shape=jax.ShapeDtypeStruct((b, length, d), emb.dtype),
        compiler_params=pltpu.CompilerParams(
            dimension_semantics=("parallel", "parallel"),
        ),
    )(shift, emb, dpe, ape)


# speedup vs baseline: 1.5587x; 1.5587x over previous
import math

import jax
import jax.numpy as jnp
from jax.experimental import pallas as pl
from jax.experimental.pallas import tpu as pltpu

DIM = 1024
HALF = DIM // 2
MAX_LEN = 5000
MID_POS = MAX_LEN // 2
BLK = 2048
SUB = 256


def _pe_kernel(shift_ref, emb_ref, dpe_ref, ape_ref, out_ref):
    i = pl.program_id(0)
    b = pl.program_id(1)
    blk_start = i * BLK

    s0 = shift_ref[b, 0]
    s1 = shift_ref[b, 1]
    s2 = shift_ref[b, 2]
    s3 = shift_ref[b, 3]
    m0 = (s0 + s1) // 2 + 1
    m1 = (s1 + s2) // 2 + 1
    m2 = (s2 + s3) // 2 + 1

    def seg_of(p):
        return (
            (p >= m0).astype(jnp.int32)
            + (p >= m1).astype(jnp.int32)
            + (p >= m2).astype(jnp.int32)
        )

    for t in range(BLK // SUB):
        rows = slice(t * SUB, (t + 1) * SUB)
        sub_start = blk_start + t * SUB
        base = MID_POS + sub_start
        j_lo = seg_of(sub_start)
        j_hi = seg_of(sub_start + SUB - 1)

        emb1 = emb_ref[0, rows, :HALF]
        second = emb_ref[0, rows, HALF:] + ape_ref[0, rows, :]

        def load_big(s):
            a = base - s
            a0 = jax.lax.div(a, 8) * 8
            return dpe_ref[pl.ds(a0, SUB + 8), :], a - a0

        @pl.when(j_lo == j_hi)
        def _single_segment(emb1=emb1, second=second, rows=rows,
                            j_lo=j_lo, load_big=load_big):
            s = shift_ref[b, j_lo]
            big, r = load_big(s)
            for rr in range(8):

                @pl.when(r == rr)
                def _(rr=rr):
                    first = emb1 + big[rr : rr + SUB]
                    out_ref[0, rows, :] = jnp.concatenate([first, second], -1)

        @pl.when(j_lo != j_hi)
        def _multi_segment(emb1=emb1, second=second, rows=rows,
                           sub_start=sub_start, load_big=load_big):
            pos = sub_start + jax.lax.broadcasted_iota(jnp.int32, (SUB, 1), 0)
            seg = seg_of(pos)
            n = SUB + 8

            def load_chunk(s):
                big, r = load_big(s)
                return pltpu.roll(big, jax.lax.rem(n - r, n), 0)[:SUB]

            sel = load_chunk(s0)
            for j, s in ((1, s1), (2, s2), (3, s3)):
                sel = jnp.where(seg == j, load_chunk(s), sel)
            out_ref[0, rows, :] = jnp.concatenate([emb1 + sel, second], -1)


def kernel(emb, shift, dpe, ape):
    b, length, d = emb.shape
    nblk = length // BLK
    grid = (nblk, b)
    return pl.pallas_call(
        _pe_kernel,
        grid=grid,
        in_specs=[
            pl.BlockSpec(memory_space=pltpu.SMEM),
            pl.BlockSpec((1, BLK, DIM), lambda i, b_: (b_, i, 0)),
            pl.BlockSpec((MAX_LEN, HALF), lambda i, b_: (0, 0)),
            pl.BlockSpec((1, BLK, HALF), lambda i, b_: (0, i, 0)),
        ],
        out_specs=pl.BlockSpec((1, BLK, DIM), lambda i, b_: (b_, i, 0)),
        out_shape=jax.ShapeDtypeStruct((b, length, d), emb.dtype),
        compiler_params=pltpu.CompilerParams(
            dimension_semantics=("parallel", "parallel"),
        ),
    )(shift, emb, dpe, ape)


# speedup vs baseline: 1.7264x; 1.1076x over previous
import math

import jax
import jax.numpy as jnp
from jax.experimental import pallas as pl
from jax.experimental.pallas import tpu as pltpu

DIM = 1024
HALF = DIM // 2
MAX_LEN = 5000
MID_POS = MAX_LEN // 2
BLK = 2048
SUB = 128


def _pe_kernel(shift_ref, emb_ref, dpe_ref, ape_ref, out_ref):
    i = pl.program_id(0)
    b = pl.program_id(1)
    blk_start = i * BLK

    s0 = shift_ref[b, 0]
    s1 = shift_ref[b, 1]
    s2 = shift_ref[b, 2]
    s3 = shift_ref[b, 3]
    m0 = (s0 + s1) // 2 + 1
    m1 = (s1 + s2) // 2 + 1
    m2 = (s2 + s3) // 2 + 1

    def seg_of(p):
        return (
            (p >= m0).astype(jnp.int32)
            + (p >= m1).astype(jnp.int32)
            + (p >= m2).astype(jnp.int32)
        )

    for t in range(BLK // SUB):
        rows = slice(t * SUB, (t + 1) * SUB)
        sub_start = blk_start + t * SUB
        base = MID_POS + sub_start
        j_lo = seg_of(sub_start)
        j_hi = seg_of(sub_start + SUB - 1)

        emb1 = emb_ref[0, rows, :HALF]
        second = emb_ref[0, rows, HALF:] + ape_ref[0, rows, :]

        def load_big(s):
            a = base - s
            a0 = jax.lax.div(a, 8) * 8
            return dpe_ref[pl.ds(a0, SUB + 8), :], a - a0

        @pl.when(j_lo == j_hi)
        def _single_segment(emb1=emb1, second=second, rows=rows,
                            j_lo=j_lo, load_big=load_big):
            s = shift_ref[b, j_lo]
            big, r = load_big(s)
            for rr in range(8):

                @pl.when(r == rr)
                def _(rr=rr):
                    first = emb1 + big[rr : rr + SUB]
                    out_ref[0, rows, :] = jnp.concatenate([first, second], -1)

        @pl.when(j_lo != j_hi)
        def _multi_segment(emb1=emb1, second=second, rows=rows,
                           sub_start=sub_start, load_big=load_big):
            pos = sub_start + jax.lax.broadcasted_iota(jnp.int32, (SUB, 1), 0)
            seg = seg_of(pos)
            n = SUB + 8

            def load_chunk(s):
                big, r = load_big(s)
                return pltpu.roll(big, jax.lax.rem(n - r, n), 0)[:SUB]

            sel = load_chunk(s0)
            for j, s in ((1, s1), (2, s2), (3, s3)):
                sel = jnp.where(seg == j, load_chunk(s), sel)
            out_ref[0, rows, :] = jnp.concatenate([emb1 + sel, second], -1)


def kernel(emb, shift, dpe, ape):
    b, length, d = emb.shape
    nblk = length // BLK
    grid = (nblk, b)
    return pl.pallas_call(
        _pe_kernel,
        grid=grid,
        in_specs=[
            pl.BlockSpec(memory_space=pltpu.SMEM),
            pl.BlockSpec((1, BLK, DIM), lambda i, b_: (b_, i, 0)),
            pl.BlockSpec((MAX_LEN, HALF), lambda i, b_: (0, 0)),
            pl.BlockSpec((1, BLK, HALF), lambda i, b_: (0, i, 0)),
        ],
        out_specs=pl.BlockSpec((1, BLK, DIM), lambda i, b_: (b_, i, 0)),
        out_shape=jax.ShapeDtypeStruct((b, length, d), emb.dtype),
        compiler_params=pltpu.CompilerParams(
            dimension_semantics=("parallel", "parallel"),
        ),
    )(shift, emb, dpe, ape)


# speedup vs baseline: 1.7278x; 1.0008x over previous
import math

import jax
import jax.numpy as jnp
from jax.experimental import pallas as pl
from jax.experimental.pallas import tpu as pltpu

DIM = 1024
HALF = DIM // 2
MAX_LEN = 5000
MID_POS = MAX_LEN // 2
BLK = 2048
SUB = 64


def _pe_kernel(shift_ref, emb_ref, dpe_ref, ape_ref, out_ref):
    i = pl.program_id(0)
    b = pl.program_id(1)
    blk_start = i * BLK

    s0 = shift_ref[b, 0]
    s1 = shift_ref[b, 1]
    s2 = shift_ref[b, 2]
    s3 = shift_ref[b, 3]
    m0 = (s0 + s1) // 2 + 1
    m1 = (s1 + s2) // 2 + 1
    m2 = (s2 + s3) // 2 + 1

    def seg_of(p):
        return (
            (p >= m0).astype(jnp.int32)
            + (p >= m1).astype(jnp.int32)
            + (p >= m2).astype(jnp.int32)
        )

    for t in range(BLK // SUB):
        rows = slice(t * SUB, (t + 1) * SUB)
        sub_start = blk_start + t * SUB
        base = MID_POS + sub_start
        j_lo = seg_of(sub_start)
        j_hi = seg_of(sub_start + SUB - 1)

        emb1 = emb_ref[0, rows, :HALF]
        second = emb_ref[0, rows, HALF:] + ape_ref[0, rows, :]

        def load_big(s):
            a = base - s
            a0 = jax.lax.div(a, 8) * 8
            return dpe_ref[pl.ds(a0, SUB + 8), :], a - a0

        @pl.when(j_lo == j_hi)
        def _single_segment(emb1=emb1, second=second, rows=rows,
                            j_lo=j_lo, load_big=load_big):
            s = shift_ref[b, j_lo]
            big, r = load_big(s)
            for rr in range(8):

                @pl.when(r == rr)
                def _(rr=rr):
                    first = emb1 + big[rr : rr + SUB]
                    out_ref[0, rows, :] = jnp.concatenate([first, second], -1)

        @pl.when(j_lo != j_hi)
        def _multi_segment(emb1=emb1, second=second, rows=rows,
                           sub_start=sub_start, load_big=load_big):
            pos = sub_start + jax.lax.broadcasted_iota(jnp.int32, (SUB, 1), 0)
            seg = seg_of(pos)
            n = SUB + 8

            def load_chunk(s):
                big, r = load_big(s)
                return pltpu.roll(big, jax.lax.rem(n - r, n), 0)[:SUB]

            sel = load_chunk(s0)
            for j, s in ((1, s1), (2, s2), (3, s3)):
                sel = jnp.where(seg == j, load_chunk(s), sel)
            out_ref[0, rows, :] = jnp.concatenate([emb1 + sel, second], -1)


def kernel(emb, shift, dpe, ape):
    b, length, d = emb.shape
    nblk = length // BLK
    grid = (nblk, b)
    return pl.pallas_call(
        _pe_kernel,
        grid=grid,
        in_specs=[
            pl.BlockSpec(memory_space=pltpu.SMEM),
            pl.BlockSpec((1, BLK, DIM), lambda i, b_: (b_, i, 0)),
            pl.BlockSpec((MAX_LEN, HALF), lambda i, b_: (0, 0)),
            pl.BlockSpec((1, BLK, HALF), lambda i, b_: (0, i, 0)),
        ],
        out_specs=pl.BlockSpec((1, BLK, DIM), lambda i, b_: (b_, i, 0)),
        out_shape=jax.ShapeDtypeStruct((b, length, d), emb.dtype),
        compiler_params=pltpu.CompilerParams(
            dimension_semantics=("parallel", "parallel"),
        ),
    )(shift, emb, dpe, ape)
